# baseline (speedup 1.0000x reference)
import functools

import jax
import jax.numpy as jnp
from jax import lax
from jax.experimental import pallas as pl
from jax.experimental.pallas import tpu as pltpu
from jax.experimental.pallas import tpu_sc as plsc

B = 64
IMG = 224
PS = 14
NPATCH = (IMG // PS) ** 2
PD = PS * PS
D = 128
NE = 16
NH = 8
DH = D // NH
HID = 256
L = B
T = NPATCH * B

NB = 16
TCH = 1024
BLK = 256
CAP = 2 * T + NE * BLK
NBLK = CAP // BLK

NC = 2
NS = 16
NW = NC * NS
TPW = T // NW
SUB = 128
NSUB = TPW // SUB


def _embed_kernel(xp_ref, wt_ref, b_ref, pos_ref, o_ref):
    xp = xp_ref[...]
    y = xp.reshape(NB * B, PD) @ wt_ref[...] + b_ref[...]
    o_ref[...] = y.reshape(NB, B, D) + pos_ref[...].reshape(NB, 1, D)


def _attn_kernel(x_ref, ipt_ref, ipb_ref, wc_ref, bc_ref, wot_ref, ob_ref,
                 o_ref, *, nb):
    x = x_ref[...].reshape(nb * L, D)
    xi = x @ ipt_ref[...] + ipb_ref[...]
    qkv = xi @ wc_ref[...] + bc_ref[...]
    hh = lax.broadcasted_iota(jnp.int32, (NH, 1, D), 0)
    dd = lax.broadcasted_iota(jnp.int32, (NH, 1, D), 2)
    msk = (dd // DH == hh).astype(jnp.float32)
    scale = 1.0 / (float(DH) ** 0.5)

    outs = []
    for i in range(nb):
        q = qkv[i * L:(i + 1) * L, 0:D]
        k = qkv[i * L:(i + 1) * L, D:2 * D]
        v = qkv[i * L:(i + 1) * L, 2 * D:3 * D]
        kp = (k[None, :, :] * msk).reshape(NH * L, D)
        s = lax.dot_general(q, kp, (((1,), (1,)), ((), ())),
                            preferred_element_type=jnp.float32)
        parts = []
        for h in range(NH):
            sh = s[:, h * L:(h + 1) * L] * scale
            m = jnp.max(sh, axis=1, keepdims=True)
            e = jnp.exp(sh - m)
            parts.append(e / jnp.sum(e, axis=1, keepdims=True))
        p = jnp.concatenate(parts, axis=1)
        vp = (v[None, :, :] * msk).reshape(NH * L, D)
        o = lax.dot_general(p, vp, (((1,), (0,)), ((), ())),
                            preferred_element_type=jnp.float32)
        outs.append(o)
    att = jnp.concatenate(outs, axis=0)
    y = att @ wot_ref[...] + ob_ref[...]
    o_ref[...] = y.reshape(nb, L, D)


def _route1_kernel(x_ref, gwt_ref, gb_ref, pk_ref, cnt_ref, carry_ref):
    i = pl.program_id(0)

    @pl.when(i == 0)
    def _():
        carry_ref[...] = jnp.zeros_like(carry_ref)

    x = x_ref[...]
    logits = x @ gwt_ref[...] + gb_ref[...]
    lm = jnp.max(logits, axis=1, keepdims=True)
    z = jnp.exp(logits - lm)
    probs = z / jnp.sum(z, axis=1, keepdims=True)
    ids = lax.broadcasted_iota(jnp.int32, (TCH, NE), 1)
    m0 = jnp.max(probs, axis=1, keepdims=True)
    e0 = jnp.min(jnp.where(probs == m0, ids, NE), axis=1, keepdims=True)
    top0 = ids == e0
    p2 = jnp.where(top0, -1.0, probs)
    m1 = jnp.max(p2, axis=1, keepdims=True)
    e1 = jnp.min(jnp.where(p2 == m1, ids, NE), axis=1, keepdims=True)
    top1 = ids == e1
    denom = m0 + m1
    w0 = m0 / denom
    w1 = m1 / denom

    oh0 = top0.astype(jnp.float32)
    oh1 = top1.astype(jnp.float32)
    oh = oh0 + oh1
    rr = lax.broadcasted_iota(jnp.int32, (TCH, TCH), 0)
    cc = lax.broadcasted_iota(jnp.int32, (TCH, TCH), 1)
    tril = (rr > cc).astype(jnp.float32)
    pre = lax.dot_general(tril, oh, (((1,), (0,)), ((), ())),
                          preferred_element_type=jnp.float32)
    carry = carry_ref[...]
    r0 = jnp.sum((pre + carry) * oh0, axis=1, keepdims=True)
    r1 = jnp.sum((pre + carry) * oh1, axis=1, keepdims=True)
    carry_ref[...] = carry + jnp.sum(oh, axis=0, keepdims=True)
    pk_ref[...] = jnp.concatenate(
        [e0.astype(jnp.float32), e1.astype(jnp.float32), w0, w1, r0, r1,
         jnp.zeros((TCH, 2), jnp.float32)], axis=1)

    @pl.when(i == pl.num_programs(0) - 1)
    def _():
        cnt_ref[...] = carry_ref[...]


def _route2_kernel(pk_ref, cnt_ref, p0_ref, p1_ref, eid_ref):
    i = pl.program_id(0)
    cnt = cnt_ref[...]
    padded = jnp.floor((cnt + (BLK - 1)) / BLK) * BLK
    ra = lax.broadcasted_iota(jnp.int32, (NE, NE), 0)
    ca = lax.broadcasted_iota(jnp.int32, (NE, NE), 1)
    trilx = (ra < ca).astype(jnp.float32)
    base = lax.dot_general(padded, trilx, (((1,), (0,)), ((), ())),
                           preferred_element_type=jnp.float32)

    pk = pk_ref[...]
    ids = lax.broadcasted_iota(jnp.int32, (TCH, NE), 1).astype(jnp.float32)
    oh0 = (ids == pk[:, 0:1]).astype(jnp.float32)
    oh1 = (ids == pk[:, 1:2]).astype(jnp.float32)
    p0 = jnp.sum(oh0 * base, axis=1, keepdims=True) + pk[:, 4:5]
    p1 = jnp.sum(oh1 * base, axis=1, keepdims=True) + pk[:, 5:6]
    p0_ref[...] = p0.astype(jnp.int32)
    p1_ref[...] = p1.astype(jnp.int32)

    @pl.when(i == 0)
    def _():
        jv = (lax.broadcasted_iota(jnp.int32, (NE, NBLK), 1)
              * BLK).astype(jnp.float32)
        baseT = jnp.reshape(base, (NE, 1))
        cmp = (jv >= baseT).astype(jnp.float32)
        eid_ref[...] = (jnp.sum(cmp, axis=0, keepdims=True)
                        - 1.0).astype(jnp.int32)


@functools.cache
def _sc_kernels():
    mesh = plsc.VectorSubcoreMesh(core_axis_name="c", subcore_axis_name="s",
                                  num_cores=NC, num_subcores=NS)

    @functools.partial(
        pl.kernel, mesh=mesh,
        out_type=jax.ShapeDtypeStruct((CAP, D), jnp.float32),
        scratch_types=[
            pltpu.VMEM((SUB, D), jnp.float32),
            pltpu.VMEM((SUB,), jnp.int32),
            pltpu.VMEM((SUB,), jnp.int32),
            pltpu.SemaphoreType.DMA,
        ],
    )
    def sc_scatter(xt_hbm, p0_hbm, p1_hbm, xs_hbm, xbuf, i0, i1, sem):
        wid = lax.axis_index("s") * NC + lax.axis_index("c")
        for it in range(NSUB):
            base = pl.multiple_of(wid * TPW + it * SUB, SUB)
            pltpu.sync_copy(xt_hbm.at[pl.ds(base, SUB)], xbuf)
            pltpu.sync_copy(p0_hbm.at[pl.ds(base, SUB)], i0)
            pltpu.sync_copy(p1_hbm.at[pl.ds(base, SUB)], i1)
            pltpu.async_copy(xbuf, xs_hbm.at[i0], sem).wait()
            pltpu.async_copy(xbuf, xs_hbm.at[i1], sem).wait()

    @functools.partial(
        pl.kernel, mesh=mesh,
        out_type=[jax.ShapeDtypeStruct((T, D), jnp.float32),
                  jax.ShapeDtypeStruct((T, D), jnp.float32)],
        scratch_types=[
            pltpu.VMEM((SUB, D), jnp.float32),
            pltpu.VMEM((SUB, D), jnp.float32),
            pltpu.VMEM((SUB,), jnp.int32),
            pltpu.VMEM((SUB,), jnp.int32),
            pltpu.SemaphoreType.DMA,
        ],
    )
    def sc_gather2(ys_hbm, p0_hbm, p1_hbm, ya_hbm, yb_hbm, bufa, bufb, i0,
                   i1, sem):
        wid = lax.axis_index("s") * NC + lax.axis_index("c")
        for it in range(NSUB):
            base = pl.multiple_of(wid * TPW + it * SUB, SUB)
            pltpu.sync_copy(p0_hbm.at[pl.ds(base, SUB)], i0)
            pltpu.sync_copy(p1_hbm.at[pl.ds(base, SUB)], i1)
            pltpu.async_copy(ys_hbm.at[i0], bufa, sem).wait()
            pltpu.async_copy(ys_hbm.at[i1], bufb, sem).wait()
            pltpu.sync_copy(bufa, ya_hbm.at[pl.ds(base, SUB)])
            pltpu.sync_copy(bufb, yb_hbm.at[pl.ds(base, SUB)])

    return sc_scatter, sc_gather2


def _sc_scatter(xt, p0f, p1f):
    return _sc_kernels()[0](xt, p0f, p1f)


def _sc_gather2(ys, p0f, p1f):
    return _sc_kernels()[1](ys, p0f, p1f)


def _expert_kernel(eid_ref, x_ref, w1_ref, b1_ref, w2_ref, b2_ref, y_ref):
    j = pl.program_id(0)
    e = eid_ref[j]
    x = x_ref[...]
    h = jnp.maximum(x @ w1_ref[e] + b1_ref[e].reshape(1, HID), 0.0)
    y_ref[...] = h @ w2_ref[e] + b2_ref[e].reshape(1, D)


def _cvec_kernel(ya_ref, yb_ref, w0_ref, w1_ref, lng_ref, lnb_ref, vwt_ref,
                 vb_ref, o_ref):
    acc = ya_ref[...] * w0_ref[...] + yb_ref[...] * w1_ref[...]
    mu = jnp.mean(acc, axis=1, keepdims=True)
    ac = acc - mu
    var = jnp.mean(ac * ac, axis=1, keepdims=True)
    y = ac * lax.rsqrt(var + 1e-5) * lng_ref[...] + lnb_ref[...]
    y = y * (1.0 / float(L))
    o_ref[...] = y @ vwt_ref[...] + vb_ref[...]


def _head_kernel(sv_ref, cwt_ref, cb_ref, g_ref, c_ref):
    i = pl.program_id(0)

    @pl.when(i == 0)
    def _():
        g_ref[...] = jnp.zeros_like(g_ref)

    g_ref[...] += jnp.sum(sv_ref[...].reshape(TCH // B, B, D), axis=0) * (
        1.0 / float(L))

    @pl.when(i == pl.num_programs(0) - 1)
    def _():
        c_ref[...] = g_ref[...] @ cwt_ref[...] + cb_ref[...]


def _moe_layer(xn, p, vec_Wt, vec_b):
    att = pl.pallas_call(
        functools.partial(_attn_kernel, nb=NB),
        grid=(NPATCH // NB,),
        in_specs=[
            pl.BlockSpec((NB, L, D), lambda i: (i, 0, 0)),
            pl.BlockSpec((D, D), lambda i: (0, 0)),
            pl.BlockSpec((1, D), lambda i: (0, 0)),
            pl.BlockSpec((D, 3 * D), lambda i: (0, 0)),
            pl.BlockSpec((1, 3 * D), lambda i: (0, 0)),
            pl.BlockSpec((D, D), lambda i: (0, 0)),
            pl.BlockSpec((1, D), lambda i: (0, 0)),
        ],
        out_specs=pl.BlockSpec((NB, L, D), lambda i: (i, 0, 0)),
        out_shape=jax.ShapeDtypeStruct((NPATCH, B, D), jnp.float32),
    )(xn, p['ip_W'].T, p['ip_b'].reshape(1, D), p['qkv_W'].T,
      p['qkv_b'].reshape(1, 3 * D), p['o_W'].T, p['o_b'].reshape(1, D))

    xt = att.reshape(T, D)

    pk, cnt = pl.pallas_call(
        _route1_kernel,
        grid=(T // TCH,),
        in_specs=[
            pl.BlockSpec((TCH, D), lambda i: (i, 0)),
            pl.BlockSpec((D, NE), lambda i: (0, 0)),
            pl.BlockSpec((1, NE), lambda i: (0, 0)),
        ],
        out_specs=[
            pl.BlockSpec((TCH, 8), lambda i: (i, 0)),
            pl.BlockSpec((1, NE), lambda i: (0, 0)),
        ],
        out_shape=[
            jax.ShapeDtypeStruct((T, 8), jnp.float32),
            jax.ShapeDtypeStruct((1, NE), jnp.float32),
        ],
        scratch_shapes=[pltpu.VMEM((1, NE), jnp.float32)],
    )(xt, p['gate_W'].T, p['gate_b'].reshape(1, NE))

    p0, p1, eid = pl.pallas_call(
        _route2_kernel,
        grid=(T // TCH,),
        in_specs=[
            pl.BlockSpec((TCH, 8), lambda i: (i, 0)),
            pl.BlockSpec((1, NE), lambda i: (0, 0)),
        ],
        out_specs=[
            pl.BlockSpec((TCH, 1), lambda i: (i, 0)),
            pl.BlockSpec((TCH, 1), lambda i: (i, 0)),
            pl.BlockSpec((1, NBLK), lambda i: (0, 0)),
        ],
        out_shape=[
            jax.ShapeDtypeStruct((T, 1), jnp.int32),
            jax.ShapeDtypeStruct((T, 1), jnp.int32),
            jax.ShapeDtypeStruct((1, NBLK), jnp.int32),
        ],
    )(pk, cnt)

    p0f = p0.reshape(T)
    p1f = p1.reshape(T)
    xs = _sc_scatter(xt, p0f, p1f)

    ys = pl.pallas_call(
        _expert_kernel,
        grid_spec=pltpu.PrefetchScalarGridSpec(
            num_scalar_prefetch=1,
            grid=(NBLK,),
            in_specs=[
                pl.BlockSpec((BLK, D), lambda j, eid: (j, 0)),
                pl.BlockSpec((NE, D, HID), lambda j, eid: (0, 0, 0)),
                pl.BlockSpec((NE, HID), lambda j, eid: (0, 0)),
                pl.BlockSpec((NE, HID, D), lambda j, eid: (0, 0, 0)),
                pl.BlockSpec((NE, D), lambda j, eid: (0, 0)),
            ],
            out_specs=pl.BlockSpec((BLK, D), lambda j, eid: (j, 0)),
        ),
        out_shape=jax.ShapeDtypeStruct((CAP, D), jnp.float32),
    )(eid.reshape(NBLK), xs, p['e_W1'], p['e_b1'], p['e_W2'], p['e_b2'])

    ya, yb = _sc_gather2(ys, p0f, p1f)

    fv = pl.pallas_call(
        _cvec_kernel,
        grid=(T // TCH,),
        in_specs=[
            pl.BlockSpec((TCH, D), lambda i: (i, 0)),
            pl.BlockSpec((TCH, D), lambda i: (i, 0)),
            pl.BlockSpec((TCH, 1), lambda i: (i, 0)),
            pl.BlockSpec((TCH, 1), lambda i: (i, 0)),
            pl.BlockSpec((1, D), lambda i: (0, 0)),
            pl.BlockSpec((1, D), lambda i: (0, 0)),
            pl.BlockSpec((D, D), lambda i: (0, 0)),
            pl.BlockSpec((1, D), lambda i: (0, 0)),
        ],
        out_specs=pl.BlockSpec((TCH, D), lambda i: (i, 0)),
        out_shape=jax.ShapeDtypeStruct((T, D), jnp.float32),
    )(ya, yb, pk[:, 2:3], pk[:, 3:4], p['ln_g'].reshape(1, D),
      p['ln_b'].reshape(1, D), vec_Wt, vec_b)
    return fv


def kernel(x, params):
    b = x.shape[0]
    xp = x.reshape(b, IMG // PS, PS, IMG // PS, PS).transpose(0, 1, 3, 2, 4)
    xp = xp.reshape(b, NPATCH, PD).transpose(1, 0, 2)

    x0 = pl.pallas_call(
        _embed_kernel,
        grid=(NPATCH // NB,),
        in_specs=[
            pl.BlockSpec((NB, B, PD), lambda i: (i, 0, 0)),
            pl.BlockSpec((PD, D), lambda i: (0, 0)),
            pl.BlockSpec((1, D), lambda i: (0, 0)),
            pl.BlockSpec((NB, D), lambda i: (i, 0)),
        ],
        out_specs=pl.BlockSpec((NB, B, D), lambda i: (i, 0, 0)),
        out_shape=jax.ShapeDtypeStruct((NPATCH, B, D), jnp.float32),
    )(xp, params['pe_W'].T, params['pe_b'].reshape(1, D),
      params['pos_emb'].reshape(NPATCH, D))

    vec_Wt = params['vec_W'].T
    vec_b = params['vec_b'].reshape(1, D)

    fv = _moe_layer(x0, params['moe1'], vec_Wt, vec_b)
    sv = _moe_layer(fv.reshape(NPATCH, B, D), params['moe2'], vec_Wt, vec_b)

    gv, cv = pl.pallas_call(
        _head_kernel,
        grid=(T // TCH,),
        in_specs=[
            pl.BlockSpec((TCH, D), lambda i: (i, 0)),
            pl.BlockSpec((D, D), lambda i: (0, 0)),
            pl.BlockSpec((1, D), lambda i: (0, 0)),
        ],
        out_specs=[
            pl.BlockSpec((B, D), lambda i: (0, 0)),
            pl.BlockSpec((B, D), lambda i: (0, 0)),
        ],
        out_shape=[
            jax.ShapeDtypeStruct((B, D), jnp.float32),
            jax.ShapeDtypeStruct((B, D), jnp.float32),
        ],
    )(sv, params['cls_W'].T, params['cls_b'].reshape(1, D))

    first_vector = fv.reshape(NPATCH, B, D).transpose(1, 0, 2)
    second_vector = sv.reshape(NPATCH, B, D).transpose(1, 0, 2)
    return (first_vector, second_vector, gv, cv)


# speedup vs baseline: 1.3947x; 1.3947x over previous
import functools

import jax
import jax.numpy as jnp
from jax import lax
from jax.experimental import pallas as pl
from jax.experimental.pallas import tpu as pltpu
from jax.experimental.pallas import tpu_sc as plsc

B = 64
IMG = 224
PS = 14
NPATCH = (IMG // PS) ** 2
PD = PS * PS
D = 128
NE = 16
NH = 8
DH = D // NH
HID = 256
L = B
T = NPATCH * B

NB = 16
TCH = 1024
BLK = 256
CAP = 2 * T + NE * BLK
NBLK = CAP // BLK

NC = 2
NS = 16
NW = NC * NS
TPW = T // NW
SUB = 128
NSUB = TPW // SUB


def _embed_kernel(xp_ref, wt_ref, b_ref, pos_ref, o_ref):
    xp = xp_ref[...]
    y = xp.reshape(NB * B, PD) @ wt_ref[...] + b_ref[...]
    o_ref[...] = y.reshape(NB, B, D) + pos_ref[...].reshape(NB, 1, D)


def _attn_kernel(x_ref, ipt_ref, ipb_ref, wc_ref, bc_ref, wot_ref, ob_ref,
                 o_ref, *, nb):
    x = x_ref[...].reshape(nb * L, D)
    xi = x @ ipt_ref[...] + ipb_ref[...]
    qkv = xi @ wc_ref[...] + bc_ref[...]
    hh = lax.broadcasted_iota(jnp.int32, (NH, 1, D), 0)
    dd = lax.broadcasted_iota(jnp.int32, (NH, 1, D), 2)
    msk = (dd // DH == hh).astype(jnp.float32)
    scale = 1.0 / (float(DH) ** 0.5)

    outs = []
    for i in range(nb):
        q = qkv[i * L:(i + 1) * L, 0:D]
        k = qkv[i * L:(i + 1) * L, D:2 * D]
        v = qkv[i * L:(i + 1) * L, 2 * D:3 * D]
        kp = (k[None, :, :] * msk).reshape(NH * L, D)
        s = lax.dot_general(q, kp, (((1,), (1,)), ((), ())),
                            preferred_element_type=jnp.float32)
        parts = []
        for h in range(NH):
            sh = s[:, h * L:(h + 1) * L] * scale
            m = jnp.max(sh, axis=1, keepdims=True)
            e = jnp.exp(sh - m)
            parts.append(e / jnp.sum(e, axis=1, keepdims=True))
        p = jnp.concatenate(parts, axis=1)
        vp = (v[None, :, :] * msk).reshape(NH * L, D)
        o = lax.dot_general(p, vp, (((1,), (0,)), ((), ())),
                            preferred_element_type=jnp.float32)
        outs.append(o)
    att = jnp.concatenate(outs, axis=0)
    y = att @ wot_ref[...] + ob_ref[...]
    o_ref[...] = y.reshape(nb, L, D)


def _route1_kernel(x_ref, gwt_ref, gb_ref, pk_ref, cnt_ref, carry_ref):
    i = pl.program_id(0)

    @pl.when(i == 0)
    def _():
        carry_ref[...] = jnp.zeros_like(carry_ref)

    x = x_ref[...]
    logits = x @ gwt_ref[...] + gb_ref[...]
    lm = jnp.max(logits, axis=1, keepdims=True)
    z = jnp.exp(logits - lm)
    probs = z / jnp.sum(z, axis=1, keepdims=True)
    ids = lax.broadcasted_iota(jnp.int32, (TCH, NE), 1)
    m0 = jnp.max(probs, axis=1, keepdims=True)
    e0 = jnp.min(jnp.where(probs == m0, ids, NE), axis=1, keepdims=True)
    top0 = ids == e0
    p2 = jnp.where(top0, -1.0, probs)
    m1 = jnp.max(p2, axis=1, keepdims=True)
    e1 = jnp.min(jnp.where(p2 == m1, ids, NE), axis=1, keepdims=True)
    top1 = ids == e1
    denom = m0 + m1
    w0 = m0 / denom
    w1 = m1 / denom

    oh0 = top0.astype(jnp.float32)
    oh1 = top1.astype(jnp.float32)
    oh = oh0 + oh1
    rr = lax.broadcasted_iota(jnp.int32, (TCH, TCH), 0)
    cc = lax.broadcasted_iota(jnp.int32, (TCH, TCH), 1)
    tril = (rr > cc).astype(jnp.float32)
    pre = lax.dot_general(tril, oh, (((1,), (0,)), ((), ())),
                          preferred_element_type=jnp.float32)
    carry = carry_ref[...]
    r0 = jnp.sum((pre + carry) * oh0, axis=1, keepdims=True)
    r1 = jnp.sum((pre + carry) * oh1, axis=1, keepdims=True)
    carry_ref[...] = carry + jnp.sum(oh, axis=0, keepdims=True)
    pk_ref[...] = jnp.concatenate(
        [e0.astype(jnp.float32), e1.astype(jnp.float32), w0, w1, r0, r1,
         jnp.zeros((TCH, 2), jnp.float32)], axis=1)

    @pl.when(i == pl.num_programs(0) - 1)
    def _():
        cnt_ref[...] = carry_ref[...]


def _route2_kernel(pk_ref, cnt_ref, p0_ref, p1_ref, eid_ref):
    i = pl.program_id(0)
    cnt = cnt_ref[...]
    padded = jnp.floor((cnt + (BLK - 1)) / BLK) * BLK
    ra = lax.broadcasted_iota(jnp.int32, (NE, NE), 0)
    ca = lax.broadcasted_iota(jnp.int32, (NE, NE), 1)
    trilx = (ra < ca).astype(jnp.float32)
    base = lax.dot_general(padded, trilx, (((1,), (0,)), ((), ())),
                           preferred_element_type=jnp.float32)

    pk = pk_ref[...]
    ids = lax.broadcasted_iota(jnp.int32, (TCH, NE), 1).astype(jnp.float32)
    oh0 = (ids == pk[:, 0:1]).astype(jnp.float32)
    oh1 = (ids == pk[:, 1:2]).astype(jnp.float32)
    p0 = jnp.sum(oh0 * base, axis=1, keepdims=True) + pk[:, 4:5]
    p1 = jnp.sum(oh1 * base, axis=1, keepdims=True) + pk[:, 5:6]
    p0_ref[...] = p0.astype(jnp.int32)
    p1_ref[...] = p1.astype(jnp.int32)

    @pl.when(i == 0)
    def _():
        jv = (lax.broadcasted_iota(jnp.int32, (NE, NBLK), 1)
              * BLK).astype(jnp.float32)
        baseT = jnp.reshape(base, (NE, 1))
        cmp = (jv >= baseT).astype(jnp.float32)
        eid_ref[...] = (jnp.sum(cmp, axis=0, keepdims=True)
                        - 1.0).astype(jnp.int32)


@functools.cache
def _sc_kernels():
    mesh = plsc.VectorSubcoreMesh(core_axis_name="c", subcore_axis_name="s",
                                  num_cores=NC, num_subcores=NS)

    @functools.partial(
        pl.kernel, mesh=mesh,
        out_type=jax.ShapeDtypeStruct((CAP, D), jnp.float32),
        scratch_types=[
            pltpu.VMEM((SUB, D), jnp.float32),
            pltpu.VMEM((SUB,), jnp.int32),
            pltpu.VMEM((SUB,), jnp.int32),
            pltpu.SemaphoreType.DMA,
        ],
    )
    def sc_scatter(xt_hbm, p0_hbm, p1_hbm, xs_hbm, xbuf, i0, i1, sem):
        wid = lax.axis_index("s") * NC + lax.axis_index("c")
        for it in range(NSUB):
            base = pl.multiple_of(wid * TPW + it * SUB, SUB)
            pltpu.sync_copy(xt_hbm.at[pl.ds(base, SUB)], xbuf)
            pltpu.sync_copy(p0_hbm.at[pl.ds(base, SUB)], i0)
            pltpu.sync_copy(p1_hbm.at[pl.ds(base, SUB)], i1)
            pltpu.async_copy(xbuf, xs_hbm.at[i0], sem).wait()
            pltpu.async_copy(xbuf, xs_hbm.at[i1], sem).wait()

    @functools.partial(
        pl.kernel, mesh=mesh,
        out_type=[jax.ShapeDtypeStruct((T, D), jnp.float32),
                  jax.ShapeDtypeStruct((T, D), jnp.float32)],
        scratch_types=[
            pltpu.VMEM((SUB, D), jnp.float32),
            pltpu.VMEM((SUB, D), jnp.float32),
            pltpu.VMEM((SUB,), jnp.int32),
            pltpu.VMEM((SUB,), jnp.int32),
            pltpu.SemaphoreType.DMA,
        ],
    )
    def sc_gather2(ys_hbm, p0_hbm, p1_hbm, ya_hbm, yb_hbm, bufa, bufb, i0,
                   i1, sem):
        wid = lax.axis_index("s") * NC + lax.axis_index("c")
        for it in range(NSUB):
            base = pl.multiple_of(wid * TPW + it * SUB, SUB)
            pltpu.sync_copy(p0_hbm.at[pl.ds(base, SUB)], i0)
            pltpu.sync_copy(p1_hbm.at[pl.ds(base, SUB)], i1)
            pltpu.async_copy(ys_hbm.at[i0], bufa, sem).wait()
            pltpu.async_copy(ys_hbm.at[i1], bufb, sem).wait()
            pltpu.sync_copy(bufa, ya_hbm.at[pl.ds(base, SUB)])
            pltpu.sync_copy(bufb, yb_hbm.at[pl.ds(base, SUB)])

    return sc_scatter, sc_gather2


def _sc_scatter(xt, p0f, p1f):
    return _sc_kernels()[0](xt, p0f, p1f)


def _sc_gather2(ys, p0f, p1f):
    return _sc_kernels()[1](ys, p0f, p1f)


def _expert_kernel(eid_ref, x_ref, w1_ref, b1_ref, w2_ref, b2_ref, y_ref):
    j = pl.program_id(0)
    e = eid_ref[j]
    x = x_ref[...]
    h = jnp.maximum(x @ w1_ref[e] + b1_ref[e].reshape(1, HID), 0.0)
    y_ref[...] = h @ w2_ref[e] + b2_ref[e].reshape(1, D)


def _cvec_kernel(ya_ref, yb_ref, w0_ref, w1_ref, lng_ref, lnb_ref, vwt_ref,
                 vb_ref, o_ref):
    acc = ya_ref[...] * w0_ref[...] + yb_ref[...] * w1_ref[...]
    mu = jnp.mean(acc, axis=1, keepdims=True)
    ac = acc - mu
    var = jnp.mean(ac * ac, axis=1, keepdims=True)
    y = ac * lax.rsqrt(var + 1e-5) * lng_ref[...] + lnb_ref[...]
    y = y * (1.0 / float(L))
    o_ref[...] = y @ vwt_ref[...] + vb_ref[...]


def _head_kernel(sv_ref, cwt_ref, cb_ref, g_ref, c_ref):
    i = pl.program_id(0)

    @pl.when(i == 0)
    def _():
        g_ref[...] = jnp.zeros_like(g_ref)

    g_ref[...] += jnp.sum(sv_ref[...].reshape(TCH // B, B, D), axis=0) * (
        1.0 / float(L))

    @pl.when(i == pl.num_programs(0) - 1)
    def _():
        c_ref[...] = g_ref[...] @ cwt_ref[...] + cb_ref[...]


def _moe_layer(xn, p, vec_Wt, vec_b):
    att = pl.pallas_call(
        functools.partial(_attn_kernel, nb=NB),
        grid=(NPATCH // NB,),
        in_specs=[
            pl.BlockSpec((NB, L, D), lambda i: (i, 0, 0)),
            pl.BlockSpec((D, D), lambda i: (0, 0)),
            pl.BlockSpec((1, D), lambda i: (0, 0)),
            pl.BlockSpec((D, 3 * D), lambda i: (0, 0)),
            pl.BlockSpec((1, 3 * D), lambda i: (0, 0)),
            pl.BlockSpec((D, D), lambda i: (0, 0)),
            pl.BlockSpec((1, D), lambda i: (0, 0)),
        ],
        out_specs=pl.BlockSpec((NB, L, D), lambda i: (i, 0, 0)),
        out_shape=jax.ShapeDtypeStruct((NPATCH, B, D), jnp.float32),
    )(xn, p['ip_W'].T, p['ip_b'].reshape(1, D), p['qkv_W'].T,
      p['qkv_b'].reshape(1, 3 * D), p['o_W'].T, p['o_b'].reshape(1, D))

    xt = att.reshape(T, D)
    return _dense_moe(xt, p, vec_Wt, vec_b)


def _gate_expert_dense(x_ref, gwt_ref, gb_ref, w1_ref, b1_ref, w2_ref,
                       b2_ref, lng_ref, lnb_ref, o_ref):
    x = x_ref[...]
    logits = x @ gwt_ref[...] + gb_ref[...]
    lm = jnp.max(logits, axis=1, keepdims=True)
    z = jnp.exp(logits - lm)
    probs = z / jnp.sum(z, axis=1, keepdims=True)
    ids = lax.broadcasted_iota(jnp.int32, (TCH, NE), 1)
    m0 = jnp.max(probs, axis=1, keepdims=True)
    e0 = jnp.min(jnp.where(probs == m0, ids, NE), axis=1, keepdims=True)
    top0 = ids == e0
    p2 = jnp.where(top0, -1.0, probs)
    m1 = jnp.max(p2, axis=1, keepdims=True)
    e1 = jnp.min(jnp.where(p2 == m1, ids, NE), axis=1, keepdims=True)
    top1 = ids == e1
    denom = m0 + m1
    w = (jnp.where(top0, m0, 0.0) + jnp.where(top1, m1, 0.0)) / denom
    acc = jnp.zeros((TCH, D), jnp.float32)
    for e in range(NE):
        h = jnp.maximum(x @ w1_ref[e] + b1_ref[e].reshape(1, HID), 0.0)
        eo = h @ w2_ref[e] + b2_ref[e].reshape(1, D)
        acc = acc + eo * w[:, e:e + 1]
    mu = jnp.mean(acc, axis=1, keepdims=True)
    ac = acc - mu
    var = jnp.mean(ac * ac, axis=1, keepdims=True)
    y = ac * lax.rsqrt(var + 1e-5) * lng_ref[...] + lnb_ref[...]
    o_ref[...] = y * (1.0 / float(L))


def _vec_only_kernel(x_ref, wt_ref, b_ref, o_ref):
    o_ref[...] = x_ref[...] @ wt_ref[...] + b_ref[...]


def _dense_moe(xt, p, vec_Wt, vec_b):
    y = pl.pallas_call(
        _gate_expert_dense,
        grid=(T // TCH,),
        in_specs=[
            pl.BlockSpec((TCH, D), lambda i: (i, 0)),
            pl.BlockSpec((D, NE), lambda i: (0, 0)),
            pl.BlockSpec((1, NE), lambda i: (0, 0)),
            pl.BlockSpec((NE, D, HID), lambda i: (0, 0, 0)),
            pl.BlockSpec((NE, HID), lambda i: (0, 0)),
            pl.BlockSpec((NE, HID, D), lambda i: (0, 0, 0)),
            pl.BlockSpec((NE, D), lambda i: (0, 0)),
            pl.BlockSpec((1, D), lambda i: (0, 0)),
            pl.BlockSpec((1, D), lambda i: (0, 0)),
        ],
        out_specs=pl.BlockSpec((TCH, D), lambda i: (i, 0)),
        out_shape=jax.ShapeDtypeStruct((T, D), jnp.float32),
    )(xt, p['gate_W'].T, p['gate_b'].reshape(1, NE), p['e_W1'], p['e_b1'],
      p['e_W2'], p['e_b2'], p['ln_g'].reshape(1, D), p['ln_b'].reshape(1, D))
    return pl.pallas_call(
        _vec_only_kernel,
        grid=(T // TCH,),
        in_specs=[
            pl.BlockSpec((TCH, D), lambda i: (i, 0)),
            pl.BlockSpec((D, D), lambda i: (0, 0)),
            pl.BlockSpec((1, D), lambda i: (0, 0)),
        ],
        out_specs=pl.BlockSpec((TCH, D), lambda i: (i, 0)),
        out_shape=jax.ShapeDtypeStruct((T, D), jnp.float32),
    )(y, vec_Wt, vec_b)


def _unused_dispatch_moe(xt, p, vec_Wt, vec_b):
    pk, cnt = pl.pallas_call(
        _route1_kernel,
        grid=(T // TCH,),
        in_specs=[
            pl.BlockSpec((TCH, D), lambda i: (i, 0)),
            pl.BlockSpec((D, NE), lambda i: (0, 0)),
            pl.BlockSpec((1, NE), lambda i: (0, 0)),
        ],
        out_specs=[
            pl.BlockSpec((TCH, 8), lambda i: (i, 0)),
            pl.BlockSpec((1, NE), lambda i: (0, 0)),
        ],
        out_shape=[
            jax.ShapeDtypeStruct((T, 8), jnp.float32),
            jax.ShapeDtypeStruct((1, NE), jnp.float32),
        ],
        scratch_shapes=[pltpu.VMEM((1, NE), jnp.float32)],
    )(xt, p['gate_W'].T, p['gate_b'].reshape(1, NE))

    p0, p1, eid = pl.pallas_call(
        _route2_kernel,
        grid=(T // TCH,),
        in_specs=[
            pl.BlockSpec((TCH, 8), lambda i: (i, 0)),
            pl.BlockSpec((1, NE), lambda i: (0, 0)),
        ],
        out_specs=[
            pl.BlockSpec((TCH, 1), lambda i: (i, 0)),
            pl.BlockSpec((TCH, 1), lambda i: (i, 0)),
            pl.BlockSpec((1, NBLK), lambda i: (0, 0)),
        ],
        out_shape=[
            jax.ShapeDtypeStruct((T, 1), jnp.int32),
            jax.ShapeDtypeStruct((T, 1), jnp.int32),
            jax.ShapeDtypeStruct((1, NBLK), jnp.int32),
        ],
    )(pk, cnt)

    p0f = p0.reshape(T)
    p1f = p1.reshape(T)
    xs = _sc_scatter(xt, p0f, p1f)

    ys = pl.pallas_call(
        _expert_kernel,
        grid_spec=pltpu.PrefetchScalarGridSpec(
            num_scalar_prefetch=1,
            grid=(NBLK,),
            in_specs=[
                pl.BlockSpec((BLK, D), lambda j, eid: (j, 0)),
                pl.BlockSpec((NE, D, HID), lambda j, eid: (0, 0, 0)),
                pl.BlockSpec((NE, HID), lambda j, eid: (0, 0)),
                pl.BlockSpec((NE, HID, D), lambda j, eid: (0, 0, 0)),
                pl.BlockSpec((NE, D), lambda j, eid: (0, 0)),
            ],
            out_specs=pl.BlockSpec((BLK, D), lambda j, eid: (j, 0)),
        ),
        out_shape=jax.ShapeDtypeStruct((CAP, D), jnp.float32),
    )(eid.reshape(NBLK), xs, p['e_W1'], p['e_b1'], p['e_W2'], p['e_b2'])

    ya, yb = _sc_gather2(ys, p0f, p1f)

    fv = pl.pallas_call(
        _cvec_kernel,
        grid=(T // TCH,),
        in_specs=[
            pl.BlockSpec((TCH, D), lambda i: (i, 0)),
            pl.BlockSpec((TCH, D), lambda i: (i, 0)),
            pl.BlockSpec((TCH, 1), lambda i: (i, 0)),
            pl.BlockSpec((TCH, 1), lambda i: (i, 0)),
            pl.BlockSpec((1, D), lambda i: (0, 0)),
            pl.BlockSpec((1, D), lambda i: (0, 0)),
            pl.BlockSpec((D, D), lambda i: (0, 0)),
            pl.BlockSpec((1, D), lambda i: (0, 0)),
        ],
        out_specs=pl.BlockSpec((TCH, D), lambda i: (i, 0)),
        out_shape=jax.ShapeDtypeStruct((T, D), jnp.float32),
    )(ya, yb, pk[:, 2:3], pk[:, 3:4], p['ln_g'].reshape(1, D),
      p['ln_b'].reshape(1, D), vec_Wt, vec_b)
    return fv


def kernel(x, params):
    b = x.shape[0]
    xp = x.reshape(b, IMG // PS, PS, IMG // PS, PS).transpose(0, 1, 3, 2, 4)
    xp = xp.reshape(b, NPATCH, PD).transpose(1, 0, 2)

    x0 = pl.pallas_call(
        _embed_kernel,
        grid=(NPATCH // NB,),
        in_specs=[
            pl.BlockSpec((NB, B, PD), lambda i: (i, 0, 0)),
            pl.BlockSpec((PD, D), lambda i: (0, 0)),
            pl.BlockSpec((1, D), lambda i: (0, 0)),
            pl.BlockSpec((NB, D), lambda i: (i, 0)),
        ],
        out_specs=pl.BlockSpec((NB, B, D), lambda i: (i, 0, 0)),
        out_shape=jax.ShapeDtypeStruct((NPATCH, B, D), jnp.float32),
    )(xp, params['pe_W'].T, params['pe_b'].reshape(1, D),
      params['pos_emb'].reshape(NPATCH, D))

    vec_Wt = params['vec_W'].T
    vec_b = params['vec_b'].reshape(1, D)

    fv = _moe_layer(x0, params['moe1'], vec_Wt, vec_b)
    sv = _moe_layer(fv.reshape(NPATCH, B, D), params['moe2'], vec_Wt, vec_b)

    gv, cv = pl.pallas_call(
        _head_kernel,
        grid=(T // TCH,),
        in_specs=[
            pl.BlockSpec((TCH, D), lambda i: (i, 0)),
            pl.BlockSpec((D, D), lambda i: (0, 0)),
            pl.BlockSpec((1, D), lambda i: (0, 0)),
        ],
        out_specs=[
            pl.BlockSpec((B, D), lambda i: (0, 0)),
            pl.BlockSpec((B, D), lambda i: (0, 0)),
        ],
        out_shape=[
            jax.ShapeDtypeStruct((B, D), jnp.float32),
            jax.ShapeDtypeStruct((B, D), jnp.float32),
        ],
    )(sv, params['cls_W'].T, params['cls_b'].reshape(1, D))

    first_vector = fv.reshape(NPATCH, B, D).transpose(1, 0, 2)
    second_vector = sv.reshape(NPATCH, B, D).transpose(1, 0, 2)
    return (first_vector, second_vector, gv, cv)
